# baseline (speedup 1.0000x reference)
import functools

import jax
import jax.numpy as jnp
from jax.experimental import pallas as pl


_BLOCK_B = 256
_NSTREAMS = 4


def _body(*refs, nsteps, scale):
    i = pl.program_id(0)
    o_ref = refs[-1]
    ns = _NSTREAMS
    y_refs = refs[:ns]
    x_refs = refs[ns:-1]
    part = jnp.float32(0.0)
    for y_ref, x_ref in zip(y_refs, x_refs):
        x = x_ref[...]
        lg = jnp.log(x)
        cols = jax.lax.broadcasted_iota(jnp.int32, x.shape, 1)
        part += jnp.sum(jnp.where(cols == y_ref[...], 0.0, lg))
    part = part.reshape(1, 1)

    @pl.when(i == 0)
    def _():
        o_ref[...] = jnp.zeros((1, 1), jnp.float32)

    o_ref[...] += part

    @pl.when(i == nsteps - 1)
    def _():
        o_ref[...] = o_ref[...] * scale


def kernel(y_pred, y):
    B, C = y_pred.shape
    bb = _BLOCK_B
    ns = _NSTREAMS
    nsteps = B // (bb * ns)
    scale = -1.0 / ((C - 1) * B)
    y2 = y.reshape(B, 1).astype(jnp.int32)

    def x_spec(s):
        return pl.BlockSpec((bb, C), lambda i, s=s: (i + s * nsteps, 0))

    def y_spec(s):
        return pl.BlockSpec((bb, 1), lambda i, s=s: (i + s * nsteps, 0))

    out = pl.pallas_call(
        functools.partial(_body, nsteps=nsteps, scale=scale),
        grid=(nsteps,),
        in_specs=[y_spec(s) for s in range(ns)] + [x_spec(s) for s in range(ns)],
        out_specs=pl.BlockSpec((1, 1), lambda i: (0, 0)),
        out_shape=jax.ShapeDtypeStruct((1, 1), jnp.float32),
    )(*([y2] * ns + [y_pred] * ns))
    return out[0, 0]


# speedup vs baseline: 1.0001x; 1.0001x over previous
import functools

import jax
import jax.numpy as jnp
from jax.experimental import pallas as pl
from jax.experimental.pallas import tpu as pltpu


_BLOCK_B = 256
_NSTREAMS = 4


def _body(*refs, nsteps, scale):
    i = pl.program_id(0)
    ns = _NSTREAMS
    y_refs = refs[:ns]
    x_refs = refs[ns : 2 * ns]
    o_ref = refs[2 * ns]
    acc_ref = refs[2 * ns + 1]

    part = None
    for y_ref, x_ref in zip(y_refs, x_refs):
        x = x_ref[...]
        lg = jnp.log(x)
        cols = jax.lax.broadcasted_iota(jnp.int32, x.shape, 1)
        p = jnp.sum(jnp.where(cols == y_ref[...], 0.0, lg), axis=0, keepdims=True)
        part = p if part is None else part + p

    @pl.when(i == 0)
    def _():
        acc_ref[...] = jnp.zeros_like(acc_ref)

    acc_ref[...] += part

    @pl.when(i == nsteps - 1)
    def _():
        o_ref[...] = jnp.sum(acc_ref[...]).reshape(1, 1) * scale


def kernel(y_pred, y):
    B, C = y_pred.shape
    bb = _BLOCK_B
    ns = _NSTREAMS
    nsteps = B // (bb * ns)
    scale = -1.0 / ((C - 1) * B)
    y2 = y.reshape(B, 1).astype(jnp.int32)

    def x_spec(s):
        return pl.BlockSpec((bb, C), lambda i, s=s: (i + s * nsteps, 0))

    def y_spec(s):
        return pl.BlockSpec((bb, 1), lambda i, s=s: (i + s * nsteps, 0))

    out = pl.pallas_call(
        functools.partial(_body, nsteps=nsteps, scale=scale),
        grid=(nsteps,),
        in_specs=[y_spec(s) for s in range(ns)] + [x_spec(s) for s in range(ns)],
        out_specs=pl.BlockSpec((1, 1), lambda i: (0, 0)),
        out_shape=jax.ShapeDtypeStruct((1, 1), jnp.float32),
        scratch_shapes=[pltpu.VMEM((1, C), jnp.float32)],
    )(*([y2] * ns + [y_pred] * ns))
    return out[0, 0]


# speedup vs baseline: 1.0010x; 1.0009x over previous
import functools

import jax
import jax.numpy as jnp
from jax.experimental import pallas as pl
from jax.experimental.pallas import tpu as pltpu


_BLOCK_B = 256
_NSTREAMS = 4


def _body(*refs, nsteps, scale):
    i = pl.program_id(0)
    ns = _NSTREAMS
    y_refs = refs[:ns]
    x_refs = refs[ns : 2 * ns]
    o_ref = refs[2 * ns]
    acc_ref = refs[2 * ns + 1]

    part = None
    for y_ref, x_ref in zip(y_refs, x_refs):
        x = x_ref[...]
        lg = jnp.log(x)
        cols = jax.lax.broadcasted_iota(jnp.int32, x.shape, 1)
        m = jnp.where(cols == y_ref[...], 0.0, lg)
        p = jnp.sum(m.reshape(m.shape[0] // 8, 8, m.shape[1]), axis=0)
        part = p if part is None else part + p

    @pl.when(i == 0)
    def _():
        acc_ref[...] = jnp.zeros_like(acc_ref)

    acc_ref[...] += part

    @pl.when(i == nsteps - 1)
    def _():
        o_ref[...] = jnp.sum(acc_ref[...]).reshape(1, 1) * scale


def kernel(y_pred, y):
    B, C = y_pred.shape
    bb = _BLOCK_B
    ns = _NSTREAMS
    nsteps = B // (bb * ns)
    scale = -1.0 / ((C - 1) * B)
    y2 = y.reshape(B, 1).astype(jnp.int32)

    def x_spec(s):
        return pl.BlockSpec((bb, C), lambda i, s=s: (i + s * nsteps, 0))

    def y_spec(s):
        return pl.BlockSpec((bb, 1), lambda i, s=s: (i + s * nsteps, 0))

    out = pl.pallas_call(
        functools.partial(_body, nsteps=nsteps, scale=scale),
        grid=(nsteps,),
        in_specs=[y_spec(s) for s in range(ns)] + [x_spec(s) for s in range(ns)],
        out_specs=pl.BlockSpec((1, 1), lambda i: (0, 0)),
        out_shape=jax.ShapeDtypeStruct((1, 1), jnp.float32),
        scratch_shapes=[pltpu.VMEM((8, C), jnp.float32)],
    )(*([y2] * ns + [y_pred] * ns))
    return out[0, 0]
